# baseline (speedup 1.0000x reference)
import jax
import jax.numpy as jnp
from jax import lax
from jax.experimental import pallas as pl
from jax.experimental.pallas import tpu as pltpu
from jax.experimental.pallas import tpu_sc as plsc

_NUM_FIELDS = 26
_VOCAB = 100000
_EMB = 16
_BATCH = 16384

_NC = 2
_NS = 16
_NW = _NC * _NS
_L = 16

_ROWS = _NUM_FIELDS * _EMB
_RPW = _ROWS // _NW
_BQ = _BATCH // 4


def _emb_kernel(tt_hbm, ft_hbm, out_hbm, row_v, feat_v, out_v, wsems):
    wid = lax.axis_index("s") * _NC + lax.axis_index("c")
    r0 = wid * _RPW

    wbs = [None, None]
    for i in range(_RPW):
        r = r0 + i
        f = r // _EMB
        if i == 0:
            pltpu.sync_copy(ft_hbm.at[f], feat_v)
        else:
            @pl.when(r % _EMB == 0)
            def _stage_feat():
                pltpu.sync_copy(ft_hbm.at[f], feat_v)
        pltpu.sync_copy(tt_hbm.at[r], row_v)
        for q in range(4):
            b0 = q * _BQ
            slot = q % 2
            if wbs[slot] is not None:
                wbs[slot].wait()

            @plsc.parallel_loop(0, _BQ, step=_L, unroll=8)
            def gather_body(off):
                out_v[slot, pl.ds(off, _L)] = plsc.load_gather(
                    row_v, [feat_v[pl.ds(b0 + off, _L)]]
                )

            wbs[slot] = pltpu.async_copy(
                out_v.at[slot], out_hbm.at[r, pl.ds(b0, _BQ)], wsems.at[slot]
            )
    for wb in wbs:
        wb.wait()


@jax.jit
def _lookup(tables_t, feats_t):
    mesh = plsc.VectorSubcoreMesh(core_axis_name="c", subcore_axis_name="s")
    return pl.kernel(
        _emb_kernel,
        out_type=jax.ShapeDtypeStruct((_ROWS, _BATCH), jnp.float32),
        mesh=mesh,
        scratch_types=[
            pltpu.VMEM((_VOCAB,), jnp.float32),
            pltpu.VMEM((_BATCH,), jnp.int32),
            pltpu.VMEM((2, _BQ), jnp.float32),
            pltpu.SemaphoreType.DMA((2,)),
        ],
        compiler_params=pltpu.CompilerParams(
            use_tc_tiling_on_sc=True, needs_layout_passes=False
        ),
    )(tables_t, feats_t)


def kernel(features, tables):
    tables_t = tables.transpose(0, 2, 1).reshape(_ROWS, _VOCAB)
    feats_t = features.T
    out_t = _lookup(tables_t, feats_t)
    return out_t.T
